# baseline (speedup 1.0000x reference)
import functools

import jax
import jax.numpy as jnp
from jax import lax
from jax.experimental import pallas as pl
from jax.experimental.pallas import tpu as pltpu
from jax.experimental.pallas import tpu_sc as plsc

E = 8
B = 128
NW = 32
LB = 16


def _routing(ids, T, K, NBLK):
    N = T * K
    CH = 128
    NCH = N // CH
    oh = (ids[:, None] == jnp.arange(E, dtype=ids.dtype)[None, :]).astype(jnp.float32)
    ohc = oh.reshape(NCH, CH, E)
    tril = jnp.tril(jnp.ones((CH, CH), jnp.float32))
    intra = jax.lax.dot_general(tril, ohc, (((1,), (1,)), ((), ())),
                                preferred_element_type=jnp.float32)
    intra = intra.transpose(1, 0, 2)
    chunk_tot = intra[:, CH - 1, :]
    excl_tril = jnp.tril(jnp.ones((NCH, NCH), jnp.float32), -1)
    chunk_pre = excl_tril @ chunk_tot
    csum = intra + chunk_pre[:, None, :]
    counts = (chunk_pre[-1] + chunk_tot[-1]).astype(jnp.int32)
    excl = (csum.reshape(N, E) - oh) * oh
    padded = ((counts + B - 1) // B) * B
    poff = jnp.concatenate([jnp.zeros((1,), jnp.int32),
                            jnp.cumsum(padded).astype(jnp.int32)])[:E]
    pos = ((excl + oh * poff[None, :].astype(jnp.float32)).sum(axis=1)
           ).astype(jnp.int32)
    pos_eo = (pos.reshape(T, K).T.reshape(K, NW, T // NW).transpose(1, 0, 2))
    blk_e = ((jnp.arange(NBLK, dtype=jnp.int32)[:, None] * B
              >= poff[None, :]).sum(axis=1) - 1).astype(jnp.int32)
    return pos_eo, blk_e


def _sc_dispatch(xb, pos_eo, *, T, K, P, d):
    tpw = T // NW
    mesh = plsc.VectorSubcoreMesh(core_axis_name="c", subcore_axis_name="s")

    @functools.partial(
        pl.kernel,
        out_type=jax.ShapeDtypeStruct((P, d), jnp.float32),
        mesh=mesh,
        scratch_types=[
            pltpu.VMEM((K, tpw), jnp.int32),
            pltpu.VMEM((tpw, d), jnp.float32),
            pltpu.SemaphoreType.DMA,
            pltpu.SemaphoreType.DMA,
        ],
    )
    def k(xb_hbm, pos_hbm, xg_hbm, pos_v, rows_v, sem0, sem1):
        wid = lax.axis_index("s") * 2 + lax.axis_index("c")
        base = wid * tpw
        cp = pltpu.async_copy(pos_hbm.at[wid], pos_v, sem0)
        cr = pltpu.async_copy(xb_hbm.at[pl.ds(base, tpw)], rows_v, sem1)
        cp.wait()
        cr.wait()
        c0 = pltpu.async_copy(rows_v, xg_hbm.at[pos_v.at[0]], sem0)
        c1 = pltpu.async_copy(rows_v, xg_hbm.at[pos_v.at[1]], sem1)
        c0.wait()
        c1.wait()

    return k(xb, pos_eo)


def _sc_combine(yg, pos_eo, *, T, K, P, d):
    tpw = T // NW
    nv = d // LB
    mesh = plsc.VectorSubcoreMesh(core_axis_name="c", subcore_axis_name="s")

    @functools.partial(
        pl.kernel,
        out_type=jax.ShapeDtypeStruct((T, d), jnp.float32),
        mesh=mesh,
        scratch_types=[
            pltpu.VMEM((K, tpw), jnp.int32),
            pltpu.VMEM((tpw, d), jnp.float32),
            pltpu.VMEM((tpw, d), jnp.float32),
            pltpu.SemaphoreType.DMA,
            pltpu.SemaphoreType.DMA,
            pltpu.SemaphoreType.DMA,
            pltpu.SemaphoreType.DMA,
            pltpu.SemaphoreType.DMA,
        ],
    )
    def k(yg_hbm, pos_hbm, out_hbm, pos_v, buf_a, buf_b,
          sa0, sb0, sa1, sb1, so):
        wid = lax.axis_index("s") * 2 + lax.axis_index("c")
        base = wid * tpw
        H = tpw // 2
        gsems = [(sa0, sb0), (sa1, sb1)]
        pltpu.sync_copy(pos_hbm.at[wid], pos_v)
        halves = []
        for h in range(2):
            r0 = h * H
            ca = pltpu.async_copy(yg_hbm.at[pos_v.at[0, pl.ds(r0, H)]],
                                  buf_a.at[pl.ds(r0, H)], gsems[h][0])
            cb = pltpu.async_copy(yg_hbm.at[pos_v.at[1, pl.ds(r0, H)]],
                                  buf_b.at[pl.ds(r0, H)], gsems[h][1])
            halves.append((ca, cb))

        def row(r, _):
            for j in range(nv):
                sl = pl.ds(j * LB, LB)
                buf_a[r, sl] = buf_a[r, sl] + buf_b[r, sl]
            return 0

        outs = []
        for h in range(2):
            r0 = h * H
            ca, cb = halves[h]
            ca.wait()
            cb.wait()
            lax.fori_loop(r0, r0 + H, row, 0)
            outs.append(pltpu.async_copy(
                buf_a.at[pl.ds(r0, H)], out_hbm.at[pl.ds(base + r0, H)], so))
        for c in outs:
            c.wait()

    return k(yg, pos_eo)


def _gemm_body(be_ref, xg_ref, w_ref, out_ref):
    xb = xg_ref[...].astype(jnp.bfloat16)
    wb = w_ref[0].astype(jnp.bfloat16)
    out_ref[...] = jnp.dot(xb, wb, preferred_element_type=jnp.float32)


def _grouped_gemm(blk_e, xg, experts_b, NBLK, d_in, d_out):
    grid_spec = pltpu.PrefetchScalarGridSpec(
        num_scalar_prefetch=1,
        grid=(NBLK,),
        in_specs=[
            pl.BlockSpec((B, d_in), lambda i, s: (i, 0)),
            pl.BlockSpec((1, d_in, d_out), lambda i, s: (s[i], 0, 0)),
        ],
        out_specs=pl.BlockSpec((B, d_out), lambda i, s: (i, 0)),
    )
    return pl.pallas_call(
        _gemm_body,
        grid_spec=grid_spec,
        out_shape=jax.ShapeDtypeStruct((NBLK * B, d_out), jnp.float32),
    )(blk_e, xg, experts_b)


def kernel(x, topk_indices, experts):
    b, s, d_in = x.shape
    d_out = experts.shape[2]
    T = b * s
    K = topk_indices.shape[-1]
    N = T * K
    NBLK = N // B + E
    P = NBLK * B

    xb = x.reshape(T, d_in)
    ids = topk_indices.reshape(N).astype(jnp.int32)

    pos_eo, blk_e = _routing(ids, T, K, NBLK)
    xg = _sc_dispatch(xb, pos_eo, T=T, K=K, P=P, d=d_in)
    yg = _grouped_gemm(blk_e, xg, experts, NBLK, d_in, d_out)
    out = _sc_combine(yg, pos_eo, T=T, K=K, P=P, d=d_out)
    return out.reshape(b, s, d_out)


# speedup vs baseline: 1.1060x; 1.1060x over previous
import functools

import jax
import jax.numpy as jnp
from jax import lax
from jax.experimental import pallas as pl
from jax.experimental.pallas import tpu as pltpu
from jax.experimental.pallas import tpu_sc as plsc

E = 8
B = 256
NW = 32
LB = 16


def _routing(ids, T, K, NBLK):
    N = T * K
    CH = 128
    NCH = N // CH
    oh = (ids[:, None] == jnp.arange(E, dtype=ids.dtype)[None, :]).astype(jnp.float32)
    ohc = oh.reshape(NCH, CH, E)
    tril = jnp.tril(jnp.ones((CH, CH), jnp.float32))
    intra = jax.lax.dot_general(tril, ohc, (((1,), (1,)), ((), ())),
                                preferred_element_type=jnp.float32)
    intra = intra.transpose(1, 0, 2)
    chunk_tot = intra[:, CH - 1, :]
    excl_tril = jnp.tril(jnp.ones((NCH, NCH), jnp.float32), -1)
    chunk_pre = excl_tril @ chunk_tot
    csum = intra + chunk_pre[:, None, :]
    counts = (chunk_pre[-1] + chunk_tot[-1]).astype(jnp.int32)
    excl = (csum.reshape(N, E) - oh) * oh
    padded = ((counts + B - 1) // B) * B
    poff = jnp.concatenate([jnp.zeros((1,), jnp.int32),
                            jnp.cumsum(padded).astype(jnp.int32)])[:E]
    pos = ((excl + oh * poff[None, :].astype(jnp.float32)).sum(axis=1)
           ).astype(jnp.int32)
    pos_eo = (pos.reshape(T, K).T.reshape(K, NW, T // NW).transpose(1, 0, 2))
    blk_e = ((jnp.arange(NBLK, dtype=jnp.int32)[:, None] * B
              >= poff[None, :]).sum(axis=1) - 1).astype(jnp.int32)
    return pos_eo, blk_e


def _sc_dispatch(xb, pos_eo, *, T, K, P, d):
    tpw = T // NW
    mesh = plsc.VectorSubcoreMesh(core_axis_name="c", subcore_axis_name="s")

    @functools.partial(
        pl.kernel,
        out_type=jax.ShapeDtypeStruct((P, d), jnp.float32),
        mesh=mesh,
        scratch_types=[
            pltpu.VMEM((K, tpw), jnp.int32),
            pltpu.VMEM((tpw, d), jnp.float32),
            pltpu.SemaphoreType.DMA,
            pltpu.SemaphoreType.DMA,
        ],
    )
    def k(xb_hbm, pos_hbm, xg_hbm, pos_v, rows_v, sem0, sem1):
        wid = lax.axis_index("s") * 2 + lax.axis_index("c")
        base = wid * tpw
        cp = pltpu.async_copy(pos_hbm.at[wid], pos_v, sem0)
        cr = pltpu.async_copy(xb_hbm.at[pl.ds(base, tpw)], rows_v, sem1)
        cp.wait()
        cr.wait()
        c0 = pltpu.async_copy(rows_v, xg_hbm.at[pos_v.at[0]], sem0)
        c1 = pltpu.async_copy(rows_v, xg_hbm.at[pos_v.at[1]], sem1)
        c0.wait()
        c1.wait()

    return k(xb, pos_eo)


def _sc_combine(yg, pos_eo, *, T, K, P, d):
    tpw = T // NW
    nv = d // LB
    mesh = plsc.VectorSubcoreMesh(core_axis_name="c", subcore_axis_name="s")

    @functools.partial(
        pl.kernel,
        out_type=jax.ShapeDtypeStruct((T, d), jnp.float32),
        mesh=mesh,
        scratch_types=[
            pltpu.VMEM((K, tpw), jnp.int32),
            pltpu.VMEM((tpw, d), jnp.float32),
            pltpu.VMEM((tpw, d), jnp.float32),
            pltpu.SemaphoreType.DMA,
            pltpu.SemaphoreType.DMA,
            pltpu.SemaphoreType.DMA,
            pltpu.SemaphoreType.DMA,
            pltpu.SemaphoreType.DMA,
        ],
    )
    def k(yg_hbm, pos_hbm, out_hbm, pos_v, buf_a, buf_b,
          sa0, sb0, sa1, sb1, so):
        wid = lax.axis_index("s") * 2 + lax.axis_index("c")
        base = wid * tpw
        H = tpw // 2
        gsems = [(sa0, sb0), (sa1, sb1)]
        pltpu.sync_copy(pos_hbm.at[wid], pos_v)
        halves = []
        for h in range(2):
            r0 = h * H
            ca = pltpu.async_copy(yg_hbm.at[pos_v.at[0, pl.ds(r0, H)]],
                                  buf_a.at[pl.ds(r0, H)], gsems[h][0])
            cb = pltpu.async_copy(yg_hbm.at[pos_v.at[1, pl.ds(r0, H)]],
                                  buf_b.at[pl.ds(r0, H)], gsems[h][1])
            halves.append((ca, cb))

        def row(r, _):
            for j in range(nv):
                sl = pl.ds(j * LB, LB)
                buf_a[r, sl] = buf_a[r, sl] + buf_b[r, sl]
            return 0

        outs = []
        for h in range(2):
            r0 = h * H
            ca, cb = halves[h]
            ca.wait()
            cb.wait()
            lax.fori_loop(r0, r0 + H, row, 0)
            outs.append(pltpu.async_copy(
                buf_a.at[pl.ds(r0, H)], out_hbm.at[pl.ds(base + r0, H)], so))
        for c in outs:
            c.wait()

    return k(yg, pos_eo)


def _gemm_body(be_ref, xg_ref, w_ref, out_ref):
    xb = xg_ref[...].astype(jnp.bfloat16)
    wb = w_ref[0].astype(jnp.bfloat16)
    out_ref[...] = jnp.dot(xb, wb, preferred_element_type=jnp.float32)


def _grouped_gemm(blk_e, xg, experts_b, NBLK, d_in, d_out):
    grid_spec = pltpu.PrefetchScalarGridSpec(
        num_scalar_prefetch=1,
        grid=(NBLK,),
        in_specs=[
            pl.BlockSpec((B, d_in), lambda i, s: (i, 0)),
            pl.BlockSpec((1, d_in, d_out), lambda i, s: (s[i], 0, 0)),
        ],
        out_specs=pl.BlockSpec((B, d_out), lambda i, s: (i, 0)),
    )
    return pl.pallas_call(
        _gemm_body,
        grid_spec=grid_spec,
        out_shape=jax.ShapeDtypeStruct((NBLK * B, d_out), jnp.float32),
    )(blk_e, xg, experts_b)


def kernel(x, topk_indices, experts):
    b, s, d_in = x.shape
    d_out = experts.shape[2]
    T = b * s
    K = topk_indices.shape[-1]
    N = T * K
    NBLK = N // B + E
    P = NBLK * B

    xb = x.reshape(T, d_in)
    ids = topk_indices.reshape(N).astype(jnp.int32)

    pos_eo, blk_e = _routing(ids, T, K, NBLK)
    xg = _sc_dispatch(xb, pos_eo, T=T, K=K, P=P, d=d_in)
    yg = _grouped_gemm(blk_e, xg, experts, NBLK, d_in, d_out)
    out = _sc_combine(yg, pos_eo, T=T, K=K, P=P, d=d_out)
    return out.reshape(b, s, d_out)
